# baseline (speedup 1.0000x reference)
import functools

import jax
import jax.numpy as jnp
import numpy as np
from jax import lax
from jax.experimental import pallas as pl
from jax.experimental.pallas import tpu as pltpu
from jax.experimental.pallas import tpu_sc as plsc

H = 128


def _emb_body(w_ref, b_ref, attr_ref, out_ref):
    a = attr_ref[...]
    out_ref[...] = (
        a[:, 0 * H:1 * H] * w_ref[0]
        + a[:, 1 * H:2 * H] * w_ref[1]
        + a[:, 2 * H:3 * H] * w_ref[2]
        + a[:, 3 * H:4 * H] * w_ref[3]
        + b_ref[0]
    )


def _node_emb(node_attr2d, conv_w, conv_b):
    n = node_attr2d.shape[0]
    nb = 1000
    grid = (n // nb,)
    return pl.pallas_call(
        _emb_body,
        grid=grid,
        in_specs=[
            pl.BlockSpec(memory_space=pltpu.SMEM),
            pl.BlockSpec(memory_space=pltpu.SMEM),
            pl.BlockSpec((nb, 4 * H), lambda i: (i, 0)),
        ],
        out_specs=pl.BlockSpec((nb, H), lambda i: (i, 0)),
        out_shape=jax.ShapeDtypeStruct((n, H), jnp.float32),
    )(conv_w, conv_b, node_attr2d)


def _gather_src_tgt(node_emb, row, col, base_e, ec):
    e = row.shape[0]
    nw = 32
    ch = 128
    n_chunks = ec // ch
    base_trips = n_chunks // nw
    extra = n_chunks - base_trips * nw

    mesh = plsc.VectorSubcoreMesh(core_axis_name="c", subcore_axis_name="s")

    @functools.partial(
        pl.kernel,
        mesh=mesh,
        out_type=(jax.ShapeDtypeStruct((ec, H // 2), jnp.int32),
                  jax.ShapeDtypeStruct((ec, H // 2), jnp.int32)),
        scratch_types=[
            pltpu.VMEM((ch,), jnp.int32),
            pltpu.VMEM((ch,), jnp.int32),
            pltpu.VMEM((2, ch, H), jnp.float32),
            pltpu.VMEM((2, ch, H), jnp.float32),
            pltpu.VMEM((ch, H // 2), jnp.int32),
            pltpu.VMEM((ch, H // 2), jnp.int32),
            pltpu.SemaphoreType.DMA,
            pltpu.SemaphoreType.DMA,
            pltpu.SemaphoreType.DMA,
            pltpu.SemaphoreType.DMA,
            pltpu.SemaphoreType.DMA,
            pltpu.SemaphoreType.DMA,
        ],
    )
    def k(emb_hbm, row_hbm, col_hbm, src_out, tgt_out, idxr_v, idxc_v,
          rows_r, rows_c, bf_r, bf_c, sem_ir, sem_ic, sem_gr, sem_gc,
          sem_sr, sem_sc):
        wid = lax.axis_index("s") * 2 + lax.axis_index("c")
        trips = base_trips + jnp.where(wid < extra, 1, 0)

        def pack_slot(s, src3, dst3):
            half = jnp.int32(0x8000)
            himask = jnp.int32(-65536)

            def body_e(ei, carry):
                for j in range(H // 32):
                    a = src3[s, ei, pl.ds(16 * j, 16)]
                    bvals = src3[s, ei, pl.ds(H // 2 + 16 * j, 16)]
                    abits = jax.lax.bitcast_convert_type(a, jnp.int32) + half
                    bbits = jax.lax.bitcast_convert_type(
                        bvals, jnp.int32) + half
                    dst3[ei, pl.ds(16 * j, 16)] = (
                        jax.lax.shift_right_logical(abits, 16)
                        | (bbits & himask))
                return carry

            lax.fori_loop(0, ch, body_e, 0)

        def pack_and_store(s, off_prev):
            pack_slot(s, rows_r, bf_r)
            pack_slot(s, rows_c, bf_c)
            pltpu.async_copy(bf_r, src_out.at[pl.ds(off_prev, ch)], sem_sr)
            pltpu.async_copy(bf_c, tgt_out.at[pl.ds(off_prev, ch)], sem_sc)

        def drain():
            pltpu.make_async_copy(bf_r, src_out.at[pl.ds(0, ch)],
                                  sem_sr).wait()
            pltpu.make_async_copy(bf_c, tgt_out.at[pl.ds(0, ch)],
                                  sem_sc).wait()

        def trip(t, b):
            off = (wid + nw * t) * ch
            hir = pltpu.async_copy(row_hbm.at[pl.ds(base_e + off, ch)],
                                   idxr_v, sem_ir)
            hic = pltpu.async_copy(col_hbm.at[pl.ds(base_e + off, ch)],
                                   idxc_v, sem_ic)
            hir.wait()
            gr = pltpu.async_copy(emb_hbm.at[idxr_v], rows_r.at[b], sem_gr)
            hic.wait()
            gc = pltpu.async_copy(emb_hbm.at[idxc_v], rows_c.at[b], sem_gc)

            pb = 1 - b

            @pl.when(t >= 2)
            def _():
                drain()

            @pl.when(t >= 1)
            def _():
                pack_and_store(pb, (wid + nw * (t - 1)) * ch)

            gr.wait()
            gc.wait()

        def body(j, carry):
            for b in range(2):
                t = 2 * j + b

                @pl.when(t < trips)
                def _():
                    trip(t, b)
            return carry

        lax.fori_loop(0, (base_trips + 2) // 2, body, 0)

        last = trips - 1
        drain()
        for s in range(2):
            @pl.when(last % 2 == s)
            def _():
                pack_and_store(s, (wid + nw * last) * ch)
        drain()

    return k(node_emb, row, col)


def _unpack_pairs(x):
    f32 = jnp.float32
    lo = jax.lax.bitcast_convert_type(jax.lax.shift_left(x, 16), f32)
    hi = jax.lax.bitcast_convert_type(x & jnp.int32(-65536), f32)
    return jnp.concatenate([lo, hi], axis=1)


def _mlp_body(src_ref, tgt_ref, edge_ref, w1st_ref, w1e_ref,
              b1_ref, w2_ref, b2_ref, w3_ref, b3_ref, out_ref):
    f32 = jnp.float32
    cat = jnp.concatenate(
        [_unpack_pairs(src_ref[...]), _unpack_pairs(tgt_ref[...])], axis=1)
    h = (
        jnp.dot(cat, w1st_ref[...], preferred_element_type=f32)
        + jnp.dot(edge_ref[...], w1e_ref[...], preferred_element_type=f32)
        + b1_ref[...]
    )
    h = jnp.maximum(h, 0.0)
    h = jnp.maximum(
        jnp.dot(h, w2_ref[...], preferred_element_type=f32) + b2_ref[...], 0.0)
    out_ref[...] = (
        jnp.dot(h, w3_ref[...], preferred_element_type=f32) + b3_ref[...])


def _mlp_body_buf(src_ref, tgt_ref, edge_ref, w1st_ref, w1e_ref, b1_ref,
                  w2_ref, b2_ref, w3_ref, b3_ref, buf_ref, out_ref):
    del buf_ref
    _mlp_body(src_ref, tgt_ref, edge_ref, w1st_ref, w1e_ref, b1_ref, w2_ref,
              b2_ref, w3_ref, b3_ref, out_ref)


def _edge_mlp_chunk(src, tgt, edge_input, w1t, b1, w2t, b2, w3t, b3, buf,
                    base_blk):
    ec = src.shape[0]
    e = edge_input.shape[0]
    eb = 2000
    nblk = ec // eb
    d1 = w1t.shape[1]
    d2 = w2t.shape[1]
    d3 = w3t.shape[1]
    loc = lambda r, c: pl.BlockSpec((r, c), lambda i: (i, 0))
    glb = lambda r, c: pl.BlockSpec((r, c), lambda i: (i + base_blk, 0))
    full = lambda r, c: pl.BlockSpec((r, c), lambda i: (0, 0))
    in_specs = [
        loc(eb, H // 2), loc(eb, H // 2), glb(eb, H),
        full(2 * H, d1), full(H, d1), full(1, d1),
        full(d1, d2), full(1, d2),
        full(d2, d3), full(1, d3),
    ]
    args = [src, tgt, edge_input,
            w1t[0:2 * H], w1t[2 * H:3 * H], b1[None, :],
            w2t, b2[None, :], w3t, b3[None, :]]
    kwargs = {}
    body = _mlp_body
    if buf is not None:
        in_specs.append(pl.BlockSpec(memory_space=pl.ANY))
        args.append(buf)
        body = _mlp_body_buf
        kwargs["input_output_aliases"] = {10: 0}
    return pl.pallas_call(
        body,
        grid=(nblk,),
        in_specs=in_specs,
        out_specs=pl.BlockSpec((eb, d3), lambda i: (i + base_blk, 0)),
        out_shape=jax.ShapeDtypeStruct((e, d3), jnp.float32),
        **kwargs,
    )(*args)


def kernel(node_attr, edge_input, edge_index, conv_w, conv_b,
           W1, b1, W2, b2, W3, b3):
    n = node_attr.shape[0]
    node_attr2d = node_attr.reshape(n, 4 * H)
    emb = _node_emb(node_attr2d, conv_w, conv_b)
    row = edge_index[0].astype(jnp.int32)
    col = edge_index[1].astype(jnp.int32)
    nch = 5
    e = row.shape[0]
    ec = e // nch
    w1t = W1.T
    pairs = [
        _gather_src_tgt(emb, row, col, c * ec, ec)
        for c in range(nch)
    ]
    out = None
    eb = 2000
    for c in range(nch):
        src, tgt = pairs[c]
        out = _edge_mlp_chunk(src, tgt, edge_input, w1t, b1, W2.T, b2, W3.T,
                              b3, out, c * (ec // eb))
    return out


# speedup vs baseline: 1.0671x; 1.0671x over previous
import functools

import jax
import jax.numpy as jnp
import numpy as np
from jax import lax
from jax.experimental import pallas as pl
from jax.experimental.pallas import tpu as pltpu
from jax.experimental.pallas import tpu_sc as plsc

H = 128


def _emb_body(w_ref, b_ref, attr_ref, out_ref):
    a = attr_ref[...]
    out_ref[...] = (
        a[:, 0 * H:1 * H] * w_ref[0]
        + a[:, 1 * H:2 * H] * w_ref[1]
        + a[:, 2 * H:3 * H] * w_ref[2]
        + a[:, 3 * H:4 * H] * w_ref[3]
        + b_ref[0]
    )


def _node_emb(node_attr2d, conv_w, conv_b):
    n = node_attr2d.shape[0]
    nb = 1000
    grid = (n // nb,)
    return pl.pallas_call(
        _emb_body,
        grid=grid,
        in_specs=[
            pl.BlockSpec(memory_space=pltpu.SMEM),
            pl.BlockSpec(memory_space=pltpu.SMEM),
            pl.BlockSpec((nb, 4 * H), lambda i: (i, 0)),
        ],
        out_specs=pl.BlockSpec((nb, H), lambda i: (i, 0)),
        out_shape=jax.ShapeDtypeStruct((n, H), jnp.float32),
    )(conv_w, conv_b, node_attr2d)


def _gather_src_tgt(node_emb, row_col, base_e, ec):
    e = row_col.shape[1]
    nw = 32
    ch = 128
    n_chunks = ec // ch
    base_trips = n_chunks // nw
    extra = n_chunks - base_trips * nw

    mesh = plsc.VectorSubcoreMesh(core_axis_name="c", subcore_axis_name="s")

    @functools.partial(
        pl.kernel,
        mesh=mesh,
        out_type=(jax.ShapeDtypeStruct((ec, H // 2), jnp.int32),
                  jax.ShapeDtypeStruct((ec, H // 2), jnp.int32)),
        scratch_types=[
            pltpu.VMEM((ch,), jnp.int32),
            pltpu.VMEM((ch,), jnp.int32),
            pltpu.VMEM((2, ch, H), jnp.float32),
            pltpu.VMEM((2, ch, H), jnp.float32),
            pltpu.VMEM((ch, H // 2), jnp.int32),
            pltpu.VMEM((ch, H // 2), jnp.int32),
            pltpu.SemaphoreType.DMA,
            pltpu.SemaphoreType.DMA,
            pltpu.SemaphoreType.DMA,
            pltpu.SemaphoreType.DMA,
            pltpu.SemaphoreType.DMA,
            pltpu.SemaphoreType.DMA,
        ],
    )
    def k(emb_hbm, ei_hbm, src_out, tgt_out, idxr_v, idxc_v,
          rows_r, rows_c, bf_r, bf_c, sem_ir, sem_ic, sem_gr, sem_gc,
          sem_sr, sem_sc):
        wid = lax.axis_index("s") * 2 + lax.axis_index("c")
        trips = base_trips + jnp.where(wid < extra, 1, 0)

        def pack_slot(s, src3, dst3):
            half = jnp.int32(0x8000)
            himask = jnp.int32(-65536)

            def body_e(ei, carry):
                for j in range(H // 32):
                    a = src3[s, ei, pl.ds(16 * j, 16)]
                    bvals = src3[s, ei, pl.ds(H // 2 + 16 * j, 16)]
                    abits = jax.lax.bitcast_convert_type(a, jnp.int32) + half
                    bbits = jax.lax.bitcast_convert_type(
                        bvals, jnp.int32) + half
                    dst3[ei, pl.ds(16 * j, 16)] = (
                        jax.lax.shift_right_logical(abits, 16)
                        | (bbits & himask))
                return carry

            lax.fori_loop(0, ch, body_e, 0)

        def pack_and_store(s, off_prev):
            pack_slot(s, rows_r, bf_r)
            pack_slot(s, rows_c, bf_c)
            pltpu.async_copy(bf_r, src_out.at[pl.ds(off_prev, ch)], sem_sr)
            pltpu.async_copy(bf_c, tgt_out.at[pl.ds(off_prev, ch)], sem_sc)

        def drain():
            pltpu.make_async_copy(bf_r, src_out.at[pl.ds(0, ch)],
                                  sem_sr).wait()
            pltpu.make_async_copy(bf_c, tgt_out.at[pl.ds(0, ch)],
                                  sem_sc).wait()

        def trip(t, b):
            off = (wid + nw * t) * ch
            hir = pltpu.async_copy(ei_hbm.at[0, pl.ds(base_e + off, ch)],
                                   idxr_v, sem_ir)
            hic = pltpu.async_copy(ei_hbm.at[1, pl.ds(base_e + off, ch)],
                                   idxc_v, sem_ic)
            hir.wait()
            gr = pltpu.async_copy(emb_hbm.at[idxr_v], rows_r.at[b], sem_gr)
            hic.wait()
            gc = pltpu.async_copy(emb_hbm.at[idxc_v], rows_c.at[b], sem_gc)

            pb = 1 - b

            @pl.when(t >= 2)
            def _():
                drain()

            @pl.when(t >= 1)
            def _():
                pack_and_store(pb, (wid + nw * (t - 1)) * ch)

            gr.wait()
            gc.wait()

        def body(j, carry):
            for b in range(2):
                t = 2 * j + b

                @pl.when(t < trips)
                def _():
                    trip(t, b)
            return carry

        lax.fori_loop(0, (base_trips + 2) // 2, body, 0)

        last = trips - 1
        drain()
        for s in range(2):
            @pl.when(last % 2 == s)
            def _():
                pack_and_store(s, (wid + nw * last) * ch)
        drain()

    return k(node_emb, row_col)


def _unpack_pairs(x):
    f32 = jnp.float32
    lo = jax.lax.bitcast_convert_type(jax.lax.shift_left(x, 16), f32)
    hi = jax.lax.bitcast_convert_type(x & jnp.int32(-65536), f32)
    return jnp.concatenate([lo, hi], axis=1)


def _mlp_body(src_ref, tgt_ref, edge_ref, w1st_ref, w1e_ref,
              b1_ref, w2_ref, b2_ref, w3_ref, b3_ref, out_ref):
    f32 = jnp.float32
    cat = jnp.concatenate(
        [_unpack_pairs(src_ref[...]), _unpack_pairs(tgt_ref[...])], axis=1)
    h = (
        jnp.dot(cat, w1st_ref[...], preferred_element_type=f32)
        + jnp.dot(edge_ref[...], w1e_ref[...], preferred_element_type=f32)
        + b1_ref[...]
    )
    h = jnp.maximum(h, 0.0)
    h = jnp.maximum(
        jnp.dot(h, w2_ref[...], preferred_element_type=f32) + b2_ref[...], 0.0)
    out_ref[...] = (
        jnp.dot(h, w3_ref[...], preferred_element_type=f32) + b3_ref[...])


def _mlp_body_buf(src_ref, tgt_ref, edge_ref, w1st_ref, w1e_ref, b1_ref,
                  w2_ref, b2_ref, w3_ref, b3_ref, buf_ref, out_ref):
    del buf_ref
    _mlp_body(src_ref, tgt_ref, edge_ref, w1st_ref, w1e_ref, b1_ref, w2_ref,
              b2_ref, w3_ref, b3_ref, out_ref)


def _edge_mlp_chunk(src, tgt, edge_input, w1t, b1, w2t, b2, w3t, b3, buf,
                    base_blk):
    ec = src.shape[0]
    e = edge_input.shape[0]
    eb = 4000
    nblk = ec // eb
    d1 = w1t.shape[1]
    d2 = w2t.shape[1]
    d3 = w3t.shape[1]
    loc = lambda r, c: pl.BlockSpec((r, c), lambda i: (i, 0))
    glb = lambda r, c: pl.BlockSpec((r, c), lambda i: (i + base_blk, 0))
    full = lambda r, c: pl.BlockSpec((r, c), lambda i: (0, 0))
    in_specs = [
        loc(eb, H // 2), loc(eb, H // 2), glb(eb, H),
        full(2 * H, d1), full(H, d1), full(1, d1),
        full(d1, d2), full(1, d2),
        full(d2, d3), full(1, d3),
    ]
    args = [src, tgt, edge_input,
            w1t[0:2 * H], w1t[2 * H:3 * H], b1[None, :],
            w2t, b2[None, :], w3t, b3[None, :]]
    kwargs = {}
    body = _mlp_body
    if buf is not None:
        in_specs.append(pl.BlockSpec(memory_space=pl.ANY))
        args.append(buf)
        body = _mlp_body_buf
        kwargs["input_output_aliases"] = {10: 0}
    return pl.pallas_call(
        body,
        grid=(nblk,),
        in_specs=in_specs,
        out_specs=pl.BlockSpec((eb, d3), lambda i: (i + base_blk, 0)),
        out_shape=jax.ShapeDtypeStruct((e, d3), jnp.float32),
        **kwargs,
    )(*args)


def kernel(node_attr, edge_input, edge_index, conv_w, conv_b,
           W1, b1, W2, b2, W3, b3):
    n = node_attr.shape[0]
    node_attr2d = node_attr.reshape(n, 4 * H)
    emb = _node_emb(node_attr2d, conv_w, conv_b)
    nch = 5
    e = edge_index.shape[1]
    ec = e // nch
    w1t = W1.T
    row_col = edge_index.astype(jnp.int32)
    pairs = [
        _gather_src_tgt(emb, row_col, c * ec, ec)
        for c in range(nch)
    ]
    out = None
    eb = 4000
    for c in range(nch):
        src, tgt = pairs[c]
        out = _edge_mlp_chunk(src, tgt, edge_input, w1t, b1, W2.T, b2, W3.T,
                              b3, out, c * (ec // eb))
    return out
